# Initial kernel scaffold; baseline (speedup 1.0000x reference)
import functools

import jax
import jax.numpy as jnp
from jax import lax
from jax.experimental import pallas as pl
from jax.experimental.pallas import tpu as pltpu
from jax.experimental.pallas import tpu_sc as plsc

N_NODES = 10000
N_EDGES = 320000
D = 128
BIAS = 1e-4

NC = 2
NS = 16
NW = NC * NS
EPW = N_EDGES // NW
LANES = 16
CHUNKS = EPW // LANES


def _node_body(f_ref, w1_ref, b1_ref, w2_ref, b2_ref, out_ref):
    x = jnp.dot(f_ref[...], w1_ref[...], preferred_element_type=jnp.float32)
    h = jnp.maximum(x + b1_ref[...], 0.0)
    wv = (w2_ref[0:1, :] + w2_ref[1:2, :]) * 0.5
    out_ref[...] = jnp.sum(h * wv, axis=1, keepdims=True) + b2_ref[0, 0] * 0.5


def _node_scores(features, W1, b1r, W2r, b2r):
    R = 2000
    return pl.pallas_call(
        _node_body,
        grid=(N_NODES // R,),
        in_specs=[
            pl.BlockSpec((R, D), lambda i: (i, 0)),
            pl.BlockSpec((D, D), lambda i: (0, 0)),
            pl.BlockSpec((1, D), lambda i: (0, 0)),
            pl.BlockSpec((2, D), lambda i: (0, 0)),
            pl.BlockSpec(memory_space=pltpu.SMEM),
        ],
        out_specs=pl.BlockSpec((R, 1), lambda i: (i, 0)),
        out_shape=jax.ShapeDtypeStruct((N_NODES, 1), jnp.float32),
    )(features, W1, b1r, W2r, b2r)


_sc_mesh = plsc.VectorSubcoreMesh(core_axis_name="c", subcore_axis_name="s")


@functools.partial(
    pl.kernel,
    mesh=_sc_mesh,
    out_type=(
        jax.ShapeDtypeStruct((N_EDGES,), jnp.float32),
        jax.ShapeDtypeStruct((N_EDGES,), jnp.float32),
    ),
    scratch_types=[
        pltpu.VMEM((N_NODES,), jnp.float32),
        pltpu.VMEM((EPW,), jnp.int32),
        pltpu.VMEM((EPW,), jnp.int32),
        pltpu.VMEM((EPW,), jnp.float32),
        pltpu.VMEM((EPW,), jnp.float32),
        pltpu.VMEM((EPW,), jnp.float32),
    ],
)
def _edge_kernel(g_hbm, src_hbm, dst_hbm, u_hbm, lp_hbm, hp_hbm,
                 g_v, src_v, dst_v, u_v, lp_v, hp_v):
    wid = lax.axis_index("s") * NC + lax.axis_index("c")
    base = wid * EPW
    pltpu.sync_copy(g_hbm, g_v)
    pltpu.sync_copy(src_hbm.at[pl.ds(base, EPW)], src_v)
    pltpu.sync_copy(dst_hbm.at[pl.ds(base, EPW)], dst_v)
    pltpu.sync_copy(u_hbm.at[pl.ds(base, EPW)], u_v)

    def body(i, carry):
        o = i * LANES
        gs = plsc.load_gather(g_v, [src_v[pl.ds(o, LANES)]])
        gd = plsc.load_gather(g_v, [dst_v[pl.ds(o, LANES)]])
        raw = gs + gd
        uu = u_v[pl.ds(o, LANES)]
        eps = uu * (2.0 * BIAS - 1.0) + (1.0 - BIAS)
        t = ((1.0 - eps) / eps) * jnp.exp(-raw)
        lp = 1.0 / (1.0 + t)
        lp_v[pl.ds(o, LANES)] = lp
        hp_v[pl.ds(o, LANES)] = 1.0 - lp
        return carry

    lax.fori_loop(0, CHUNKS, body, 0)
    pltpu.sync_copy(lp_v, lp_hbm.at[pl.ds(base, EPW)])
    pltpu.sync_copy(hp_v, hp_hbm.at[pl.ds(base, EPW)])


def kernel(features, edges, u, W1, b1, W2, b2):
    g = _node_scores(features, W1, b1.reshape(1, D), W2.reshape(2, D),
                     b2.reshape(1, 1))
    lp, hp = _edge_kernel(g.reshape(-1), edges[0], edges[1], u)
    return (lp, hp)


# speedup vs baseline: 26.2122x; 26.2122x over previous
import functools

import jax
import jax.numpy as jnp
from jax import lax
from jax.experimental import pallas as pl
from jax.experimental.pallas import tpu as pltpu
from jax.experimental.pallas import tpu_sc as plsc

N_NODES = 10000
N_EDGES = 320000
D = 128
BIAS = 1e-4

NC = 2
NS = 16
NW = NC * NS
EPW = N_EDGES // NW
LANES = 16
CHUNKS = EPW // LANES


def _node_body(f_ref, w1_ref, b1_ref, w2_ref, b2_ref, out_ref):
    x = jnp.dot(f_ref[...], w1_ref[...], preferred_element_type=jnp.float32)
    h = jnp.maximum(x + b1_ref[...], 0.0)
    wv = (w2_ref[0:1, :] + w2_ref[1:2, :]) * 0.5
    out_ref[...] = jnp.sum(h * wv, axis=1, keepdims=True) + b2_ref[0, 0] * 0.5


def _node_scores(features, W1, b1r, W2r, b2r):
    R = 2000
    return pl.pallas_call(
        _node_body,
        grid=(N_NODES // R,),
        in_specs=[
            pl.BlockSpec((R, D), lambda i: (i, 0)),
            pl.BlockSpec((D, D), lambda i: (0, 0)),
            pl.BlockSpec((1, D), lambda i: (0, 0)),
            pl.BlockSpec((2, D), lambda i: (0, 0)),
            pl.BlockSpec(memory_space=pltpu.SMEM),
        ],
        out_specs=pl.BlockSpec((R, 1), lambda i: (i, 0)),
        out_shape=jax.ShapeDtypeStruct((N_NODES, 1), jnp.float32),
    )(features, W1, b1r, W2r, b2r)


@functools.cache
def _make_edge_kernel():
    mesh = plsc.VectorSubcoreMesh(core_axis_name="c", subcore_axis_name="s")

    @functools.partial(
        pl.kernel,
        mesh=mesh,
        compiler_params=pltpu.CompilerParams(needs_layout_passes=False),
        out_type=(
            jax.ShapeDtypeStruct((N_EDGES,), jnp.float32),
            jax.ShapeDtypeStruct((N_EDGES,), jnp.float32),
        ),
        scratch_types=[
            pltpu.VMEM((N_NODES,), jnp.float32),
            pltpu.VMEM((EPW,), jnp.int32),
            pltpu.VMEM((EPW,), jnp.int32),
            pltpu.VMEM((EPW,), jnp.float32),
            pltpu.VMEM((EPW,), jnp.float32),
            pltpu.VMEM((EPW,), jnp.float32),
        ],
    )
    def edge_kernel(g_hbm, src_hbm, dst_hbm, u_hbm, lp_hbm, hp_hbm,
                    g_v, src_v, dst_v, u_v, lp_v, hp_v):
        wid = lax.axis_index("s") * NC + lax.axis_index("c")
        base = wid * EPW
        pltpu.sync_copy(g_hbm, g_v)
        pltpu.sync_copy(src_hbm.at[pl.ds(base, EPW)], src_v)
        pltpu.sync_copy(dst_hbm.at[pl.ds(base, EPW)], dst_v)
        pltpu.sync_copy(u_hbm.at[pl.ds(base, EPW)], u_v)

        def body(i, carry):
            o = i * LANES
            gs = plsc.load_gather(g_v, [src_v[pl.ds(o, LANES)]])
            gd = plsc.load_gather(g_v, [dst_v[pl.ds(o, LANES)]])
            raw = gs + gd
            uu = u_v[pl.ds(o, LANES)]
            eps = uu * (2.0 * BIAS - 1.0) + (1.0 - BIAS)
            t = ((1.0 - eps) / eps) * jnp.exp(-raw)
            lp = 1.0 / (1.0 + t)
            lp_v[pl.ds(o, LANES)] = lp
            hp_v[pl.ds(o, LANES)] = 1.0 - lp
            return carry

        lax.fori_loop(0, CHUNKS, body, 0)
        pltpu.sync_copy(lp_v, lp_hbm.at[pl.ds(base, EPW)])
        pltpu.sync_copy(hp_v, hp_hbm.at[pl.ds(base, EPW)])

    return edge_kernel


def kernel(features, edges, u, W1, b1, W2, b2):
    g = _node_scores(features, W1, b1.reshape(1, D), W2.reshape(2, D),
                     b2.reshape(1, 1))
    lp, hp = _make_edge_kernel()(g.reshape(-1), edges[0], edges[1], u)
    return (lp, hp)


# speedup vs baseline: 52.3123x; 1.9957x over previous
import functools

import jax
import jax.numpy as jnp
from jax import lax
from jax.experimental import pallas as pl
from jax.experimental.pallas import tpu as pltpu
from jax.experimental.pallas import tpu_sc as plsc

N_NODES = 10000
N_EDGES = 320000
D = 128
BIAS = 1e-4

NC = 2
NS = 16
NW = NC * NS
EPW = N_EDGES // NW
LANES = 16

GRID = 5
G_PAD = 10240
RN = G_PAD // GRID
RE = N_EDGES // GRID


def _node_body(f_ref, w1_ref, b1_ref, w2_ref, b2_ref, e_ref,
               g_ref, src_ref, dst_ref):
    i = pl.program_id(0)
    x = jnp.dot(f_ref[...], w1_ref[...], preferred_element_type=jnp.float32)
    h = jnp.maximum(x + b1_ref[...], 0.0)
    wv = (w2_ref[0:1, :] + w2_ref[1:2, :]) * 0.5
    g_ref[pl.ds(i * RN, RN)] = jnp.sum(h * wv, axis=1) + b2_ref[0, 0] * 0.5
    src_ref[pl.ds(i * RE, RE)] = e_ref[0, :]
    dst_ref[pl.ds(i * RE, RE)] = e_ref[1, :]


def _node_scores(features, W1, b1r, W2r, b2r, edges):
    return pl.pallas_call(
        _node_body,
        grid=(GRID,),
        in_specs=[
            pl.BlockSpec((RN, D), lambda i: (i, 0)),
            pl.BlockSpec((D, D), lambda i: (0, 0)),
            pl.BlockSpec((1, D), lambda i: (0, 0)),
            pl.BlockSpec((2, D), lambda i: (0, 0)),
            pl.BlockSpec(memory_space=pltpu.SMEM),
            pl.BlockSpec((2, RE), lambda i: (0, i)),
        ],
        out_specs=[
            pl.BlockSpec((G_PAD,), lambda i: (0,)),
            pl.BlockSpec((N_EDGES,), lambda i: (0,)),
            pl.BlockSpec((N_EDGES,), lambda i: (0,)),
        ],
        out_shape=[
            jax.ShapeDtypeStruct((G_PAD,), jnp.float32),
            jax.ShapeDtypeStruct((N_EDGES,), jnp.int32),
            jax.ShapeDtypeStruct((N_EDGES,), jnp.int32),
        ],
    )(features, W1, b1r, W2r, b2r, edges)


@functools.cache
def _make_edge_kernel():
    mesh = plsc.VectorSubcoreMesh(core_axis_name="c", subcore_axis_name="s")

    @functools.partial(
        pl.kernel,
        mesh=mesh,
        compiler_params=pltpu.CompilerParams(needs_layout_passes=False),
        out_type=(
            jax.ShapeDtypeStruct((N_EDGES,), jnp.float32),
            jax.ShapeDtypeStruct((N_EDGES,), jnp.float32),
        ),
        scratch_types=[
            pltpu.VMEM((G_PAD,), jnp.float32),
            pltpu.VMEM((EPW,), jnp.int32),
            pltpu.VMEM((EPW,), jnp.int32),
            pltpu.VMEM((EPW,), jnp.float32),
            pltpu.VMEM((EPW,), jnp.float32),
            pltpu.VMEM((EPW,), jnp.float32),
            pltpu.SemaphoreType.DMA,
        ],
    )
    def edge_kernel(g_hbm, src_hbm, dst_hbm, u_hbm, lp_hbm, hp_hbm,
                    g_v, src_v, dst_v, u_v, lp_v, hp_v, sem):
        wid = lax.axis_index("s") * NC + lax.axis_index("c")
        base = wid * EPW
        c1 = pltpu.async_copy(g_hbm, g_v, sem)
        c2 = pltpu.async_copy(src_hbm.at[pl.ds(base, EPW)], src_v, sem)
        c3 = pltpu.async_copy(dst_hbm.at[pl.ds(base, EPW)], dst_v, sem)
        c4 = pltpu.async_copy(u_hbm.at[pl.ds(base, EPW)], u_v, sem)
        c1.wait()
        c2.wait()
        c3.wait()
        c4.wait()

        @plsc.parallel_loop(0, EPW, step=LANES, unroll=8)
        def _(o):
            gs = plsc.load_gather(g_v, [src_v[pl.ds(o, LANES)]])
            gd = plsc.load_gather(g_v, [dst_v[pl.ds(o, LANES)]])
            uu = u_v[pl.ds(o, LANES)]
            p = uu * (1.0 - BIAS * 2.0)
            num = p + BIAS
            den = (1.0 - BIAS) - p
            e = jnp.exp(-(gs + gd))
            lp = den / (den + num * e)
            lp_v[pl.ds(o, LANES)] = lp
            hp_v[pl.ds(o, LANES)] = 1.0 - lp

        o1 = pltpu.async_copy(lp_v, lp_hbm.at[pl.ds(base, EPW)], sem)
        o2 = pltpu.async_copy(hp_v, hp_hbm.at[pl.ds(base, EPW)], sem)
        o1.wait()
        o2.wait()

    return edge_kernel


def kernel(features, edges, u, W1, b1, W2, b2):
    g, src, dst = _node_scores(features, W1, b1.reshape(1, D),
                               W2.reshape(2, D), b2.reshape(1, 1), edges)
    lp, hp = _make_edge_kernel()(g, src, dst, u)
    return (lp, hp)


# speedup vs baseline: 53.9313x; 1.0309x over previous
import functools

import jax
import jax.numpy as jnp
from jax import lax
from jax.experimental import pallas as pl
from jax.experimental.pallas import tpu as pltpu
from jax.experimental.pallas import tpu_sc as plsc

N_NODES = 10000
N_EDGES = 320000
D = 128
BIAS = 1e-4

NC = 2
NS = 16
NW = NC * NS
LANES = 16
EPW = 10112
LAST_BASE = N_EDGES - EPW

GRID = 5
G_PAD = 10240
RN = G_PAD // GRID


def _node_body(f_ref, w1_ref, b1_ref, w2_ref, b2_ref, g_ref):
    i = pl.program_id(0)
    x = jnp.dot(f_ref[...], w1_ref[...], preferred_element_type=jnp.float32)
    h = jnp.maximum(x + b1_ref[...], 0.0)
    wv = (w2_ref[0:1, :] + w2_ref[1:2, :]) * 0.5
    g_ref[pl.ds(i * RN, RN)] = jnp.sum(h * wv, axis=1) + b2_ref[0, 0] * 0.5


def _node_scores(features, W1, b1r, W2r, b2r):
    return pl.pallas_call(
        _node_body,
        grid=(GRID,),
        in_specs=[
            pl.BlockSpec((RN, D), lambda i: (i, 0)),
            pl.BlockSpec((D, D), lambda i: (0, 0)),
            pl.BlockSpec((1, D), lambda i: (0, 0)),
            pl.BlockSpec((2, D), lambda i: (0, 0)),
            pl.BlockSpec(memory_space=pltpu.SMEM),
        ],
        out_specs=pl.BlockSpec((G_PAD,), lambda i: (0,)),
        out_shape=jax.ShapeDtypeStruct((G_PAD,), jnp.float32),
    )(features, W1, b1r, W2r, b2r)


@functools.cache
def _make_edge_kernel():
    mesh = plsc.VectorSubcoreMesh(core_axis_name="c", subcore_axis_name="s")

    @functools.partial(
        pl.kernel,
        mesh=mesh,
        compiler_params=pltpu.CompilerParams(needs_layout_passes=False),
        out_type=(
            jax.ShapeDtypeStruct((N_EDGES,), jnp.float32),
            jax.ShapeDtypeStruct((N_EDGES,), jnp.float32),
        ),
        scratch_types=[
            pltpu.VMEM((G_PAD,), jnp.float32),
            pltpu.VMEM((2, EPW), jnp.int32),
            pltpu.VMEM((EPW,), jnp.float32),
            pltpu.VMEM((EPW,), jnp.float32),
            pltpu.VMEM((EPW,), jnp.float32),
            pltpu.SemaphoreType.DMA,
        ],
    )
    def edge_kernel(g_hbm, e_hbm, u_hbm, lp_hbm, hp_hbm,
                    g_v, ed_v, u_v, lp_v, hp_v, sem):
        wid = lax.axis_index("s") * NC + lax.axis_index("c")
        base = jnp.minimum(wid * EPW, LAST_BASE)
        c1 = pltpu.async_copy(g_hbm, g_v, sem)
        c2 = pltpu.async_copy(e_hbm.at[:, pl.ds(base, EPW)], ed_v, sem)
        c4 = pltpu.async_copy(u_hbm.at[pl.ds(base, EPW)], u_v, sem)
        c1.wait()
        c2.wait()
        c4.wait()

        @plsc.parallel_loop(0, EPW, step=LANES, unroll=8)
        def _(o):
            gs = plsc.load_gather(g_v, [ed_v[0, pl.ds(o, LANES)]])
            gd = plsc.load_gather(g_v, [ed_v[1, pl.ds(o, LANES)]])
            uu = u_v[pl.ds(o, LANES)]
            p = uu * (1.0 - BIAS * 2.0)
            num = p + BIAS
            den = (1.0 - BIAS) - p
            e = jnp.exp(-(gs + gd))
            lp = den / (den + num * e)
            lp_v[pl.ds(o, LANES)] = lp
            hp_v[pl.ds(o, LANES)] = 1.0 - lp

        o1 = pltpu.async_copy(lp_v, lp_hbm.at[pl.ds(base, EPW)], sem)
        o2 = pltpu.async_copy(hp_v, hp_hbm.at[pl.ds(base, EPW)], sem)
        o1.wait()
        o2.wait()

    return edge_kernel


def kernel(features, edges, u, W1, b1, W2, b2):
    g = _node_scores(features, W1, b1.reshape(1, D),
                     W2.reshape(2, D), b2.reshape(1, 1))
    lp, hp = _make_edge_kernel()(g, edges, u)
    return (lp, hp)


# speedup vs baseline: 57.5518x; 1.0671x over previous
import functools

import jax
import jax.numpy as jnp
from jax import lax
from jax.experimental import pallas as pl
from jax.experimental.pallas import tpu as pltpu
from jax.experimental.pallas import tpu_sc as plsc

N_NODES = 10000
N_EDGES = 320000
D = 128
BIAS = 1e-4

NC = 2
NS = 16
NW = NC * NS
LANES = 16
EPW = 10112
LAST_BASE = N_EDGES - EPW

GRID = 5
G_PAD = 10240
RN = G_PAD // GRID


def _node_body(f_ref, w1_ref, b1_ref, w2_ref, b2_ref, g_ref):
    i = pl.program_id(0)
    x = jnp.dot(f_ref[...], w1_ref[...], preferred_element_type=jnp.float32)
    h = jnp.maximum(x + b1_ref[...], 0.0)
    wv = (w2_ref[0:1, :] + w2_ref[1:2, :]) * 0.5
    gv = lax.dot_general(wv, h, (((1,), (1,)), ((), ())),
                         preferred_element_type=jnp.float32)
    g_ref[pl.ds(i * RN, RN)] = gv[0] + b2_ref[0, 0] * 0.5


def _node_scores(features, W1, b1r, W2r, b2r):
    return pl.pallas_call(
        _node_body,
        grid=(GRID,),
        in_specs=[
            pl.BlockSpec((RN, D), lambda i: (i, 0)),
            pl.BlockSpec((D, D), lambda i: (0, 0)),
            pl.BlockSpec((1, D), lambda i: (0, 0)),
            pl.BlockSpec((2, D), lambda i: (0, 0)),
            pl.BlockSpec(memory_space=pltpu.SMEM),
        ],
        out_specs=pl.BlockSpec((G_PAD,), lambda i: (0,)),
        out_shape=jax.ShapeDtypeStruct((G_PAD,), jnp.float32),
    )(features, W1, b1r, W2r, b2r)


@functools.cache
def _make_edge_kernel():
    mesh = plsc.VectorSubcoreMesh(core_axis_name="c", subcore_axis_name="s")

    @functools.partial(
        pl.kernel,
        mesh=mesh,
        compiler_params=pltpu.CompilerParams(needs_layout_passes=False),
        out_type=(
            jax.ShapeDtypeStruct((N_EDGES,), jnp.float32),
            jax.ShapeDtypeStruct((N_EDGES,), jnp.float32),
        ),
        scratch_types=[
            pltpu.VMEM((G_PAD,), jnp.float32),
            pltpu.VMEM((2, EPW), jnp.int32),
            pltpu.VMEM((EPW,), jnp.float32),
            pltpu.VMEM((EPW,), jnp.float32),
            pltpu.VMEM((EPW,), jnp.float32),
            pltpu.SemaphoreType.DMA,
        ],
    )
    def edge_kernel(g_hbm, e_hbm, u_hbm, lp_hbm, hp_hbm,
                    g_v, ed_v, u_v, lp_v, hp_v, sem):
        wid = lax.axis_index("s") * NC + lax.axis_index("c")
        base = jnp.minimum(wid * EPW, LAST_BASE)
        c1 = pltpu.async_copy(g_hbm, g_v, sem)
        c2 = pltpu.async_copy(e_hbm.at[:, pl.ds(base, EPW)], ed_v, sem)
        c4 = pltpu.async_copy(u_hbm.at[pl.ds(base, EPW)], u_v, sem)
        c1.wait()
        c2.wait()
        c4.wait()

        @plsc.parallel_loop(0, EPW, step=LANES, unroll=8)
        def _(o):
            gs = plsc.load_gather(g_v, [ed_v[0, pl.ds(o, LANES)]])
            gd = plsc.load_gather(g_v, [ed_v[1, pl.ds(o, LANES)]])
            uu = u_v[pl.ds(o, LANES)]
            p = uu * (1.0 - BIAS * 2.0)
            num = p + BIAS
            den = (1.0 - BIAS) - p
            e = jnp.exp(-(gs + gd))
            lp = den / (den + num * e)
            lp_v[pl.ds(o, LANES)] = lp
            hp_v[pl.ds(o, LANES)] = 1.0 - lp

        o1 = pltpu.async_copy(lp_v, lp_hbm.at[pl.ds(base, EPW)], sem)
        o2 = pltpu.async_copy(hp_v, hp_hbm.at[pl.ds(base, EPW)], sem)
        o1.wait()
        o2.wait()

    return edge_kernel


def kernel(features, edges, u, W1, b1, W2, b2):
    g = _node_scores(features, W1, b1.reshape(1, D),
                     W2.reshape(2, D), b2.reshape(1, 1))
    lp, hp = _make_edge_kernel()(g, edges, u)
    return (lp, hp)
